# Initial kernel scaffold; baseline (speedup 1.0000x reference)
import jax
import jax.numpy as jnp
from jax.experimental import pallas as pl


def kernel(seq_len, table):
    raise NotImplementedError("write your pallas kernel here")


# speedup vs baseline: 7.0542x; 7.0542x over previous
import jax
import jax.numpy as jnp
from jax import lax
from jax.experimental import pallas as pl
from jax.experimental.pallas import tpu as pltpu
from jax.experimental.pallas import tpu_sc as plsc

_MAX_DIST = 1024
_PROJ_DIM = 32
_SEQ_LEN = 2048

_NC = 2
_NS = 16
_NW = _NC * _NS
_RPT = _SEQ_LEN // _NW
_WLEN = _SEQ_LEN + _RPT - 1
_CH = 128
_NCHUNK = (_WLEN + _CH - 1) // _CH
_WPAD = _NCHUNK * _CH
_GROUP = 8


def _body(table_hbm, out_hbm, idx_v, win_v, sem):
    w = lax.axis_index("c") * _NS + lax.axis_index("s")
    base = (_MAX_DIST - 1) - (_RPT * w + _RPT - 1)

    def build_idx(k, carry):
        c = k // (_CH // 16)
        q = k % (_CH // 16)
        v = lax.iota(jnp.int32, 16) + (base + k * 16)
        v = jnp.minimum(jnp.maximum(v, 0), 2 * _MAX_DIST - 2)
        idx_v[c, pl.ds(q * 16, 16)] = v
        return carry

    lax.fori_loop(0, _WPAD // 16, build_idx, 0)

    gathers = [
        pltpu.async_copy(
            table_hbm.at[idx_v.at[c]],
            win_v.at[pl.ds(c * _CH, _CH)],
            sem,
        )
        for c in range(_NCHUNK)
    ]
    for g in gathers:
        g.wait()

    row0 = _RPT * w

    def out_group(g, carry):
        handles = []
        for r in range(_GROUP):
            l = g * _GROUP + r
            handles.append(
                pltpu.async_copy(
                    win_v.at[pl.ds(_RPT - 1 - l, _SEQ_LEN)],
                    out_hbm.at[row0 + l],
                    sem,
                )
            )
        for h in handles:
            h.wait()
        return carry

    lax.fori_loop(0, _RPT // _GROUP, out_group, 0)


def kernel(seq_len, table):
    del seq_len
    run = pl.kernel(
        _body,
        mesh=plsc.VectorSubcoreMesh(core_axis_name="c", subcore_axis_name="s"),
        out_type=jax.ShapeDtypeStruct((_SEQ_LEN, _SEQ_LEN, _PROJ_DIM), jnp.float32),
        scratch_types=[
            pltpu.VMEM((_NCHUNK, _CH), jnp.int32),
            pltpu.VMEM((_WPAD, _PROJ_DIM), jnp.float32),
            pltpu.SemaphoreType.DMA,
        ],
        compiler_params=pltpu.CompilerParams(use_tc_tiling_on_sc=False),
    )
    return run(table)


# speedup vs baseline: 12.2676x; 1.7390x over previous
import jax
import jax.numpy as jnp
from jax import lax
from jax.experimental import pallas as pl
from jax.experimental.pallas import tpu as pltpu
from jax.experimental.pallas import tpu_sc as plsc

_MAX_DIST = 1024
_PROJ_DIM = 32
_SEQ_LEN = 2048
_TROWS = 2 * _MAX_DIST - 1
_ROW_F = _SEQ_LEN * _PROJ_DIM

_NC = 2
_NS = 16
_NW = _NC * _NS
_RPT = _SEQ_LEN // _NW
_LO = 960
_WBUF_ROWS = _LO + _MAX_DIST + _TROWS
_GROUP = 8


def _body(table_hbm, out_hbm, win_v, sem):
    w = lax.axis_index("c") * _NS + lax.axis_index("s")
    base = (_MAX_DIST - 1) - (_RPT * w + _RPT - 1)
    toff = (_LO - base) * _PROJ_DIM

    pltpu.async_copy(
        table_hbm, win_v.at[pl.ds(toff, _TROWS * _PROJ_DIM)], sem
    ).wait()

    h = jnp.maximum(0, -base)
    t = jnp.maximum(0, base + _RPT)
    first0 = win_v[pl.ds(toff, 16)]
    first1 = win_v[pl.ds(toff + 16, 16)]
    last_off = toff + (_TROWS - 1) * _PROJ_DIM
    last0 = win_v[pl.ds(last_off, 16)]
    last1 = win_v[pl.ds(last_off + 16, 16)]

    def fill_head(m, carry):
        o = (_LO + m) * _PROJ_DIM
        win_v[pl.ds(o, 16)] = first0
        win_v[pl.ds(o + 16, 16)] = first1
        return carry

    def fill_tail(m, carry):
        o = (_LO + _SEQ_LEN + _RPT - 2 - m) * _PROJ_DIM
        win_v[pl.ds(o, 16)] = last0
        win_v[pl.ds(o + 16, 16)] = last1
        return carry

    lax.fori_loop(0, h, fill_head, 0)
    lax.fori_loop(0, t, fill_tail, 0)

    row0 = _RPT * w

    def out_group(g, carry):
        handles = []
        for r in range(_GROUP):
            l = g * _GROUP + r
            src = (_LO + _RPT - 1 - l) * _PROJ_DIM
            handles.append(
                pltpu.async_copy(
                    win_v.at[pl.ds(src, _ROW_F)],
                    out_hbm.at[row0 + l],
                    sem,
                )
            )
        for hd in handles:
            hd.wait()
        return carry

    lax.fori_loop(0, _RPT // _GROUP, out_group, 0)


def kernel(seq_len, table):
    del seq_len
    run = pl.kernel(
        _body,
        mesh=plsc.VectorSubcoreMesh(core_axis_name="c", subcore_axis_name="s"),
        out_type=jax.ShapeDtypeStruct((_SEQ_LEN, _ROW_F), jnp.float32),
        scratch_types=[
            pltpu.VMEM((_WBUF_ROWS * _PROJ_DIM,), jnp.float32),
            pltpu.SemaphoreType.DMA,
        ],
        compiler_params=pltpu.CompilerParams(use_tc_tiling_on_sc=False),
    )
    flat = run(table.reshape(_TROWS * _PROJ_DIM))
    return flat.reshape(_SEQ_LEN, _SEQ_LEN, _PROJ_DIM)
